# Initial kernel scaffold; baseline (speedup 1.0000x reference)
import jax
import jax.numpy as jnp
from jax.experimental import pallas as pl


def kernel(h, x, edges, edge_attr, params):
    raise NotImplementedError("write your pallas kernel here")


# speedup vs baseline: 3.0693x; 3.0693x over previous
import functools

import jax
import jax.numpy as jnp
from jax import lax
from jax.experimental import pallas as pl
from jax.experimental.pallas import tpu as pltpu
from jax.experimental.pallas import tpu_sc as plsc

F32 = jnp.float32
XP = 16
BE = 2560
BN = 2000
CH = 400
CX = 2000


def _silu(v):
    return v * jax.nn.sigmoid(v)


def _init_body(h_ref, we_ref, be_ref, w1a_ref, w1b_ref, b1_ref,
               wee_ref, w1e_ref, bee_ref,
               h0_ref, hw1_ref, hw2_ref, wfea_ref, bfea_ref):
    h0 = jnp.dot(h_ref[...], we_ref[...], preferred_element_type=F32) + be_ref[...]
    h0_ref[...] = h0
    hw1_ref[...] = jnp.dot(h0, w1a_ref[...], preferred_element_type=F32) + b1_ref[...]
    hw2_ref[...] = jnp.dot(h0, w1b_ref[...], preferred_element_type=F32)
    wfea_ref[...] = jnp.dot(wee_ref[...], w1e_ref[...], preferred_element_type=F32)
    bfea_ref[...] = jnp.dot(bee_ref[...], w1e_ref[...], preferred_element_type=F32)


def _tc_init(h, we, be, w1a, w1b, b1, wee, w1e, bee, n, hid):
    grid = (n // BN,)
    blk = lambda r, c: pl.BlockSpec((r, c), lambda i: (i, 0))
    const = lambda r, c: pl.BlockSpec((r, c), lambda i: (0, 0))
    return pl.pallas_call(
        _init_body,
        grid=grid,
        in_specs=[blk(BN, hid), const(hid, hid), const(1, hid),
                  const(hid, hid), const(hid, hid), const(1, hid),
                  const(16, hid), const(hid, hid), const(1, hid)],
        out_specs=[blk(BN, hid), blk(BN, hid), blk(BN, hid),
                   const(16, hid), const(1, hid)],
        out_shape=[jax.ShapeDtypeStruct((n, hid), F32),
                   jax.ShapeDtypeStruct((n, hid), F32),
                   jax.ShapeDtypeStruct((n, hid), F32),
                   jax.ShapeDtypeStruct((16, hid), F32),
                   jax.ShapeDtypeStruct((1, hid), F32)],
    )(h, we, be, w1a, w1b, b1, wee, w1e, bee)


def _edge_body(first, last, *refs):
    n_wts = 5 + (9 if first else 8) + (2 if last else 0)
    ins, outs = refs[:n_wts], refs[n_wts:]
    ghr_ref, ghc_ref, e_ref, xr_ref, xc_ref = ins[:5]
    w = list(ins[5:])
    if first:
        w1e_ref, bfea_ref = w[0], w[1]
        w = w[2:]
    else:
        w1e_ref = w[0]
        w = w[1:]
    wr_ref, w2_ref, b2_ref, wc1_ref, bc1_ref, wc2_ref, bc2_ref = w[:7]
    diff = xr_ref[...] - xc_ref[...]
    radial = jnp.sum(diff * diff, axis=1, keepdims=True)
    z = (ghr_ref[...] + ghc_ref[...] + radial * wr_ref[...]
         + jnp.dot(e_ref[...], w1e_ref[...], preferred_element_type=F32))
    if first:
        z = z + bfea_ref[...]
    m1 = _silu(z)
    m = _silu(jnp.dot(m1, w2_ref[...], preferred_element_type=F32) + b2_ref[...])
    c1 = _silu(jnp.dot(m, wc1_ref[...], preferred_element_type=F32) + bc1_ref[...])
    phi = jnp.sum(c1 * wc2_ref[...], axis=1, keepdims=True) + bc2_ref[:, :1]
    dn = diff / (jnp.sqrt(radial + 1e-8) + 1.0)
    outs[0][...] = m
    outs[1][...] = dn * phi
    if last:
        weo_ref, beo_ref = w[7], w[8]
        outs[2][...] = jnp.dot(m, weo_ref[...], preferred_element_type=F32) + beo_ref[...]


def _tc_edge(first, last, ghr, ghc, e, xr, xc, wts, e_cnt, hid):
    grid = (e_cnt // BE,)
    blk = lambda r, c: pl.BlockSpec((r, c), lambda i: (i, 0))
    const = lambda r, c: pl.BlockSpec((r, c), lambda i: (0, 0))
    e_w = e.shape[1]
    in_specs = [blk(BE, hid), blk(BE, hid), blk(BE, e_w), blk(BE, XP), blk(BE, XP)]
    in_specs += [const(w.shape[0], w.shape[1]) for w in wts]
    out_specs = [blk(BE, hid), blk(BE, XP)]
    out_shape = [jax.ShapeDtypeStruct((e_cnt, hid), F32),
                 jax.ShapeDtypeStruct((e_cnt, XP), F32)]
    if last:
        out_specs.append(blk(BE, 16))
        out_shape.append(jax.ShapeDtypeStruct((e_cnt, 16), F32))
    body = functools.partial(_edge_body, first, last)
    return pl.pallas_call(
        body, grid=grid, in_specs=in_specs, out_specs=out_specs,
        out_shape=out_shape,
    )(ghr, ghc, e, xr, xc, *wts)


def _node_body(final, *refs):
    (h_ref, agg_ref, dx_ref, xp_ref,
     wn1h_ref, wn1a0_ref, wn1a1_ref, bn1_ref, wn2_ref, bn2_ref,
     wa_ref, wb_ref, bx_ref, *outs) = refs
    u = _silu(jnp.dot(h_ref[...], wn1h_ref[...], preferred_element_type=F32)
              + jnp.dot(agg_ref[0], wn1a0_ref[...], preferred_element_type=F32)
              + jnp.dot(agg_ref[1], wn1a1_ref[...], preferred_element_type=F32)
              + bn1_ref[...])
    hn = h_ref[...] + jnp.dot(u, wn2_ref[...], preferred_element_type=F32) + bn2_ref[...]
    outs[1][...] = xp_ref[...] + dx_ref[0] + dx_ref[1]
    if final:
        outs[0][...] = jnp.dot(hn, wa_ref[...], preferred_element_type=F32) + bx_ref[...]
    else:
        outs[0][...] = hn
        outs[2][...] = jnp.dot(hn, wa_ref[...], preferred_element_type=F32) + bx_ref[...]
        outs[3][...] = jnp.dot(hn, wb_ref[...], preferred_element_type=F32)


def _tc_node(final, h, agg, dx, xp, wts, n, hid):
    grid = (n // BN,)
    blk = lambda r, c: pl.BlockSpec((r, c), lambda i: (i, 0))
    const = lambda r, c: pl.BlockSpec((r, c), lambda i: (0, 0))
    in_specs = [blk(BN, hid),
                pl.BlockSpec((2, BN, hid // 2), lambda i: (0, i, 0)),
                pl.BlockSpec((2, BN, XP), lambda i: (0, i, 0)),
                blk(BN, XP)]
    in_specs += [const(w.shape[0], w.shape[1]) for w in wts]
    if final:
        out_specs = [blk(BN, hid), blk(BN, XP)]
        out_shape = [jax.ShapeDtypeStruct((n, hid), F32),
                     jax.ShapeDtypeStruct((n, XP), F32)]
    else:
        out_specs = [blk(BN, hid), blk(BN, XP), blk(BN, hid), blk(BN, hid)]
        out_shape = [jax.ShapeDtypeStruct((n, hid), F32),
                     jax.ShapeDtypeStruct((n, XP), F32),
                     jax.ShapeDtypeStruct((n, hid), F32),
                     jax.ShapeDtypeStruct((n, hid), F32)]
    body = functools.partial(_node_body, final)
    return pl.pallas_call(
        body, grid=grid, in_specs=in_specs, out_specs=out_specs,
        out_shape=out_shape,
    )(h, agg, dx, xp, *wts)


def _sc_gather(hw1, hw2, xp, row, col, n, e_cnt, hid):
    info = plsc.get_sparse_core_info()
    nc, ns = info.num_cores, info.num_subcores
    nw = nc * ns
    per_w = e_cnt // nw
    mesh = plsc.VectorSubcoreMesh(core_axis_name="c", subcore_axis_name="s")

    @functools.partial(
        pl.kernel, mesh=mesh,
        compiler_params=pltpu.CompilerParams(use_tc_tiling_on_sc=False),
        out_type=(jax.ShapeDtypeStruct((e_cnt, hid), F32),
                  jax.ShapeDtypeStruct((e_cnt, hid), F32),
                  jax.ShapeDtypeStruct((e_cnt, XP), F32),
                  jax.ShapeDtypeStruct((e_cnt, XP), F32)),
        scratch_types=[pltpu.VMEM((CH,), jnp.int32),
                       pltpu.VMEM((CX,), jnp.int32),
                       pltpu.VMEM((CH, hid), F32),
                       pltpu.VMEM((CX, XP), F32),
                       pltpu.SemaphoreType.DMA],
    )
    def k(hw1_h, hw2_h, xp_h, row_h, col_h, ghr_h, ghc_h, gxr_h, gxc_h,
          idxh_v, idxx_v, r128_v, r16_v, sem):
        wid = lax.axis_index("s") * nc + lax.axis_index("c")
        base = wid * per_w

        def h_loop(idx_h, tab_h, out_h):
            def body(c, _):
                off = base + c * CH
                pltpu.sync_copy(idx_h.at[pl.ds(off, CH)], idxh_v)
                pltpu.async_copy(tab_h.at[idxh_v], r128_v, sem).wait()
                pltpu.sync_copy(r128_v, out_h.at[pl.ds(off, CH)])
                return 0
            lax.fori_loop(0, per_w // CH, body, 0)

        def x_loop(idx_h, out_h):
            def body(c, _):
                off = base + c * CX
                pltpu.sync_copy(idx_h.at[pl.ds(off, CX)], idxx_v)
                pltpu.async_copy(xp_h.at[idxx_v], r16_v, sem).wait()
                pltpu.sync_copy(r16_v, out_h.at[pl.ds(off, CX)])
                return 0
            lax.fori_loop(0, per_w // CX, body, 0)

        h_loop(row_h, hw1_h, ghr_h)
        h_loop(col_h, hw2_h, ghc_h)
        x_loop(row_h, gxr_h)
        x_loop(col_h, gxc_h)

    return k(hw1, hw2, xp, row, col)


def _sc_scatter(m, trans, row, zeros_h, zeros_x, n, e_cnt, hid):
    info = plsc.get_sparse_core_info()
    nc, ns = info.num_cores, info.num_subcores
    hh = hid // nc
    per_w = e_cnt // ns
    half_e = e_cnt // nc
    per_w_t = half_e // ns
    npad = zeros_h.shape[0]
    rows_per_tile = npad // ns
    mesh = plsc.VectorSubcoreMesh(core_axis_name="c", subcore_axis_name="s")

    @functools.partial(
        pl.kernel, mesh=mesh,
        compiler_params=pltpu.CompilerParams(use_tc_tiling_on_sc=False),
        out_type=(jax.ShapeDtypeStruct((nc, npad, hh), F32),
                  jax.ShapeDtypeStruct((nc, npad, XP), F32)),
        scratch_types=[pltpu.VMEM((CH,), jnp.int32),
                       pltpu.VMEM((CH, hh), F32),
                       pltpu.VMEM((CH, XP), F32),
                       pltpu.VMEM_SHARED((npad, hh), F32),
                       pltpu.VMEM_SHARED((npad, XP), F32)],
    )
    def k(m_h, t_h, row_h, z_h, zx_h, agg_h, dx_h,
          idx_v, m_v, t_v, acc_s, accx_s):
        cid = lax.axis_index("c")
        sid = lax.axis_index("s")

        @pl.when(sid == 0)
        def _zero():
            pltpu.sync_copy(z_h, acc_s)
            pltpu.sync_copy(zx_h, accx_s)

        plsc.subcore_barrier()

        def body(c, _):
            off = sid * per_w + c * CH
            pltpu.sync_copy(row_h.at[pl.ds(off, CH)], idx_v)
            pltpu.sync_copy(m_h.at[pl.ds(off, CH), pl.ds(cid * hh, hh)], m_v)
            pltpu.sync_copy(m_v, acc_s.at[idx_v], add=True)
            return 0
        lax.fori_loop(0, per_w // CH, body, 0)

        def body_t(c, _):
            off = cid * half_e + sid * per_w_t + c * CH
            pltpu.sync_copy(row_h.at[pl.ds(off, CH)], idx_v)
            pltpu.sync_copy(t_h.at[pl.ds(off, CH)], t_v)
            pltpu.sync_copy(t_v, accx_s.at[idx_v], add=True)
            return 0
        lax.fori_loop(0, per_w_t // CH, body_t, 0)

        plsc.subcore_barrier()

        r0 = sid * rows_per_tile
        pltpu.sync_copy(acc_s.at[pl.ds(r0, rows_per_tile)],
                        agg_h.at[cid, pl.ds(r0, rows_per_tile)])
        pltpu.sync_copy(accx_s.at[pl.ds(r0, rows_per_tile)],
                        dx_h.at[cid, pl.ds(r0, rows_per_tile)])

    return k(m, trans, row, zeros_h, zeros_x)


def kernel(h, x, edges, edge_attr, params):
    n, in_node = h.shape
    e_cnt = edges.shape[1]
    hid = params['emb_node'][0].shape[1]
    row = edges[0].astype(jnp.int32)
    col = edges[1].astype(jnp.int32)
    xp = jnp.zeros((n, XP), F32).at[:, :3].set(x)
    npad = ((n + 127) // 128) * 128
    zeros_h = jnp.zeros((npad, hid // 2), F32)
    zeros_x = jnp.zeros((npad, XP), F32)

    layers = params['layers']

    def split_w1(lp):
        w1, b1 = lp['edge_mlp1']
        return (w1[:hid], w1[hid:2 * hid], w1[2 * hid:2 * hid + 1],
                w1[2 * hid + 1:], b1.reshape(1, hid))

    w1a0, w1b0, wr0, w1e0, b10 = split_w1(layers[0])
    we, be = params['emb_node']
    wee, bee = params['emb_edge']

    h0, hw1, hw2, wfea, bfea = _tc_init(
        h, we, be.reshape(1, hid), w1a0, w1b0, b10,
        wee, w1e0, bee.reshape(1, hid), n, hid)

    hcur, xpcur = h0, xp
    e_feat = edge_attr
    for li, lp in enumerate(layers):
        first = li == 0
        last = li == len(layers) - 1
        w1a, w1b, wr, w1e, b1 = split_w1(lp)
        w2, b2 = lp['edge_mlp2']
        wc1, bc1 = lp['coord_mlp1']
        wc2, bc2 = lp['coord_mlp2']
        ghr, ghc, gxr, gxc = _sc_gather(hw1, hw2, xpcur, row, col, n, e_cnt, hid)
        bc2v = jnp.broadcast_to(bc2.reshape(1, 1), (1, hid))
        if first:
            wts = [wfea, bfea, wr, w2, b2.reshape(1, hid),
                   wc1, bc1.reshape(1, hid), wc2.reshape(1, hid), bc2v]
        else:
            wts = [w1e, wr, w2, b2.reshape(1, hid),
                   wc1, bc1.reshape(1, hid), wc2.reshape(1, hid), bc2v]
        if last:
            weo, beo = params['emb_edge_out']
            wts += [weo, beo.reshape(1, 16)]
        eouts = _tc_edge(first, last, ghr, ghc, e_feat, gxr, gxc, wts, e_cnt, hid)
        m = eouts[0]
        trans = eouts[1]
        agg, dx = _sc_scatter(m, trans, row, zeros_h, zeros_x, n, e_cnt, hid)
        wn1, bn1 = lp['node_mlp1']
        wn2, bn2 = lp['node_mlp2']
        hh = hid // 2
        if last:
            who, bho = params['emb_node_out']
            nwts = [wn1[:hid], wn1[hid:hid + hh], wn1[hid + hh:], bn1.reshape(1, hid),
                    wn2, bn2.reshape(1, hid), who, who, bho.reshape(1, in_node)]
            hout, xpcur = _tc_node(True, hcur, agg, dx, xpcur, nwts, n, hid)
        else:
            w1a_n, w1b_n, _, _, b1_n = split_w1(layers[li + 1])
            nwts = [wn1[:hid], wn1[hid:hid + hh], wn1[hid + hh:], bn1.reshape(1, hid),
                    wn2, bn2.reshape(1, hid), w1a_n, w1b_n, b1_n]
            hcur, xpcur, hw1, hw2 = _tc_node(False, hcur, agg, dx, xpcur, nwts, n, hid)
        e_feat = m

    e_out = eouts[2]
    return (hout, xpcur[:, :3], e_out)


# speedup vs baseline: 3.3637x; 1.0959x over previous
import functools

import jax
import jax.numpy as jnp
from jax import lax
from jax.experimental import pallas as pl
from jax.experimental.pallas import tpu as pltpu
from jax.experimental.pallas import tpu_sc as plsc

F32 = jnp.float32
XP = 16
BE = 2560
BN = 2000
CH = 400
CX = 2000


def _silu(v):
    return v * jax.nn.sigmoid(v)


def _init_body(h_ref, we_ref, be_ref, w1a_ref, w1b_ref, b1_ref,
               wee_ref, w1e_ref, bee_ref,
               h0_ref, hw1_ref, hw2_ref, wfea_ref, bfea_ref):
    h0 = jnp.dot(h_ref[...], we_ref[...], preferred_element_type=F32) + be_ref[...]
    h0_ref[...] = h0
    hw1_ref[...] = jnp.dot(h0, w1a_ref[...], preferred_element_type=F32) + b1_ref[...]
    hw2_ref[...] = jnp.dot(h0, w1b_ref[...], preferred_element_type=F32)
    wfea_ref[...] = jnp.dot(wee_ref[...], w1e_ref[...], preferred_element_type=F32)
    bfea_ref[...] = jnp.dot(bee_ref[...], w1e_ref[...], preferred_element_type=F32)


def _tc_init(h, we, be, w1a, w1b, b1, wee, w1e, bee, n, hid):
    grid = (n // BN,)
    blk = lambda r, c: pl.BlockSpec((r, c), lambda i: (i, 0))
    const = lambda r, c: pl.BlockSpec((r, c), lambda i: (0, 0))
    return pl.pallas_call(
        _init_body,
        grid=grid,
        in_specs=[blk(BN, hid), const(hid, hid), const(1, hid),
                  const(hid, hid), const(hid, hid), const(1, hid),
                  const(16, hid), const(hid, hid), const(1, hid)],
        out_specs=[blk(BN, hid), blk(BN, hid), blk(BN, hid),
                   const(16, hid), const(1, hid)],
        out_shape=[jax.ShapeDtypeStruct((n, hid), F32),
                   jax.ShapeDtypeStruct((n, hid), F32),
                   jax.ShapeDtypeStruct((n, hid), F32),
                   jax.ShapeDtypeStruct((16, hid), F32),
                   jax.ShapeDtypeStruct((1, hid), F32)],
    )(h, we, be, w1a, w1b, b1, wee, w1e, bee)


def _edge_body(first, last, *refs):
    n_wts = 5 + (9 if first else 8) + (2 if last else 0)
    ins, outs = refs[:n_wts], refs[n_wts:]
    ghr_ref, ghc_ref, e_ref, xr_ref, xc_ref = ins[:5]
    w = list(ins[5:])
    if first:
        w1e_ref, bfea_ref = w[0], w[1]
        w = w[2:]
    else:
        w1e_ref = w[0]
        w = w[1:]
    wr_ref, w2_ref, b2_ref, wc1_ref, bc1_ref, wc2_ref, bc2_ref = w[:7]
    diff = xr_ref[...] - xc_ref[...]
    radial = jnp.sum(diff * diff, axis=1, keepdims=True)
    z = (ghr_ref[...] + ghc_ref[...] + radial * wr_ref[...]
         + jnp.dot(e_ref[...], w1e_ref[...], preferred_element_type=F32))
    if first:
        z = z + bfea_ref[...]
    m1 = _silu(z)
    m = _silu(jnp.dot(m1, w2_ref[...], preferred_element_type=F32) + b2_ref[...])
    c1 = _silu(jnp.dot(m, wc1_ref[...], preferred_element_type=F32) + bc1_ref[...])
    phi = jnp.sum(c1 * wc2_ref[...], axis=1, keepdims=True) + bc2_ref[:, :1]
    dn = diff / (jnp.sqrt(radial + 1e-8) + 1.0)
    outs[0][...] = m
    outs[1][...] = dn * phi
    if last:
        weo_ref, beo_ref = w[7], w[8]
        outs[2][...] = jnp.dot(m, weo_ref[...], preferred_element_type=F32) + beo_ref[...]


def _tc_edge(first, last, ghr, ghc, e, xr, xc, wts, e_cnt, hid):
    grid = (e_cnt // BE,)
    blk = lambda r, c: pl.BlockSpec((r, c), lambda i: (i, 0))
    const = lambda r, c: pl.BlockSpec((r, c), lambda i: (0, 0))
    e_w = e.shape[1]
    in_specs = [blk(BE, hid), blk(BE, hid), blk(BE, e_w), blk(BE, XP), blk(BE, XP)]
    in_specs += [const(w.shape[0], w.shape[1]) for w in wts]
    out_specs = [blk(BE, hid), blk(BE, XP)]
    out_shape = [jax.ShapeDtypeStruct((e_cnt, hid), F32),
                 jax.ShapeDtypeStruct((e_cnt, XP), F32)]
    if last:
        out_specs.append(blk(BE, 16))
        out_shape.append(jax.ShapeDtypeStruct((e_cnt, 16), F32))
    body = functools.partial(_edge_body, first, last)
    return pl.pallas_call(
        body, grid=grid, in_specs=in_specs, out_specs=out_specs,
        out_shape=out_shape,
    )(ghr, ghc, e, xr, xc, *wts)


def _node_body(final, *refs):
    (h_ref, agg_ref, dx_ref, xp_ref,
     wn1h_ref, wn1a0_ref, wn1a1_ref, bn1_ref, wn2_ref, bn2_ref,
     wa_ref, wb_ref, bx_ref, *outs) = refs
    u = _silu(jnp.dot(h_ref[...], wn1h_ref[...], preferred_element_type=F32)
              + jnp.dot(agg_ref[0], wn1a0_ref[...], preferred_element_type=F32)
              + jnp.dot(agg_ref[1], wn1a1_ref[...], preferred_element_type=F32)
              + bn1_ref[...])
    hn = h_ref[...] + jnp.dot(u, wn2_ref[...], preferred_element_type=F32) + bn2_ref[...]
    outs[1][...] = xp_ref[...] + dx_ref[0] + dx_ref[1]
    if final:
        outs[0][...] = jnp.dot(hn, wa_ref[...], preferred_element_type=F32) + bx_ref[...]
    else:
        outs[0][...] = hn
        outs[2][...] = jnp.dot(hn, wa_ref[...], preferred_element_type=F32) + bx_ref[...]
        outs[3][...] = jnp.dot(hn, wb_ref[...], preferred_element_type=F32)


def _tc_node(final, h, agg, dx, xp, wts, n, hid):
    grid = (n // BN,)
    blk = lambda r, c: pl.BlockSpec((r, c), lambda i: (i, 0))
    const = lambda r, c: pl.BlockSpec((r, c), lambda i: (0, 0))
    in_specs = [blk(BN, hid),
                pl.BlockSpec((2, BN, hid // 2), lambda i: (0, i, 0)),
                pl.BlockSpec((2, BN, XP), lambda i: (0, i, 0)),
                blk(BN, XP)]
    in_specs += [const(w.shape[0], w.shape[1]) for w in wts]
    if final:
        out_specs = [blk(BN, hid), blk(BN, XP)]
        out_shape = [jax.ShapeDtypeStruct((n, hid), F32),
                     jax.ShapeDtypeStruct((n, XP), F32)]
    else:
        out_specs = [blk(BN, hid), blk(BN, XP), blk(BN, hid), blk(BN, hid)]
        out_shape = [jax.ShapeDtypeStruct((n, hid), F32),
                     jax.ShapeDtypeStruct((n, XP), F32),
                     jax.ShapeDtypeStruct((n, hid), F32),
                     jax.ShapeDtypeStruct((n, hid), F32)]
    body = functools.partial(_node_body, final)
    return pl.pallas_call(
        body, grid=grid, in_specs=in_specs, out_specs=out_specs,
        out_shape=out_shape,
    )(h, agg, dx, xp, *wts)


def _pipe_gather(idx_h, tab_h, out_h, base, n_chunks, ch,
                 idx2, row2, gs, os_):
    def step(c, buf):
        oth = 1 - buf

        @pl.when(c >= 2)
        def _():
            pltpu.make_async_copy(
                row2.at[buf], out_h.at[pl.ds(base + (c - 2) * ch, ch)],
                os_[buf]).wait()

        pltpu.sync_copy(idx_h.at[pl.ds(base + c * ch, ch)], idx2.at[buf])
        pltpu.async_copy(tab_h.at[idx2.at[buf]], row2.at[buf], gs[buf])

        @pl.when(c >= 1)
        def _():
            pltpu.make_async_copy(tab_h.at[idx2.at[oth]], row2.at[oth],
                                  gs[oth]).wait()
            pltpu.async_copy(row2.at[oth],
                             out_h.at[pl.ds(base + (c - 1) * ch, ch)],
                             os_[oth])

    def body(c, _):
        @pl.when(c % 2 == 0)
        def _():
            step(c, 0)

        @pl.when(c % 2 == 1)
        def _():
            step(c, 1)
        return 0

    lax.fori_loop(0, n_chunks, body, 0)
    lastb = (n_chunks - 1) % 2
    pltpu.make_async_copy(tab_h.at[idx2.at[lastb]], row2.at[lastb],
                          gs[lastb]).wait()
    pltpu.async_copy(row2.at[lastb],
                     out_h.at[pl.ds(base + (n_chunks - 1) * ch, ch)],
                     os_[lastb])
    pltpu.make_async_copy(
        row2.at[1 - lastb], out_h.at[pl.ds(base + (n_chunks - 2) * ch, ch)],
        os_[1 - lastb]).wait()
    pltpu.make_async_copy(
        row2.at[lastb], out_h.at[pl.ds(base + (n_chunks - 1) * ch, ch)],
        os_[lastb]).wait()


def _sc_gather(hw1, hw2, xp, row, col, n, e_cnt, hid):
    info = plsc.get_sparse_core_info()
    nc, ns = info.num_cores, info.num_subcores
    nw = nc * ns
    per_w = e_cnt // nw
    kk = per_w // CH
    mesh = plsc.VectorSubcoreMesh(core_axis_name="c", subcore_axis_name="s")

    @functools.partial(
        pl.kernel, mesh=mesh,
        compiler_params=pltpu.CompilerParams(use_tc_tiling_on_sc=False),
        out_type=(jax.ShapeDtypeStruct((e_cnt, hid), F32),
                  jax.ShapeDtypeStruct((e_cnt, hid), F32),
                  jax.ShapeDtypeStruct((e_cnt, XP), F32),
                  jax.ShapeDtypeStruct((e_cnt, XP), F32)),
        scratch_types=[pltpu.VMEM((2, CH), jnp.int32),
                       pltpu.VMEM((2, CH, hid), F32),
                       pltpu.VMEM((2, CH, XP), F32),
                       pltpu.SemaphoreType.DMA,
                       pltpu.SemaphoreType.DMA,
                       pltpu.SemaphoreType.DMA,
                       pltpu.SemaphoreType.DMA],
    )
    def k(hw1_h, hw2_h, xp_h, row_h, col_h, ghr_h, ghc_h, gxr_h, gxc_h,
          idx2, r128_2, r16_2, g0, g1, o0, o1):
        wid = lax.axis_index("s") * nc + lax.axis_index("c")
        base = wid * per_w
        gs, os_ = (g0, g1), (o0, o1)
        _pipe_gather(row_h, hw1_h, ghr_h, base, kk, CH, idx2, r128_2, gs, os_)
        _pipe_gather(col_h, hw2_h, ghc_h, base, kk, CH, idx2, r128_2, gs, os_)
        _pipe_gather(row_h, xp_h, gxr_h, base, kk, CH, idx2, r16_2, gs, os_)
        _pipe_gather(col_h, xp_h, gxc_h, base, kk, CH, idx2, r16_2, gs, os_)

    return k(hw1, hw2, xp, row, col)


def _sc_scatter(m, trans, row, zeros_h, zeros_x, n, e_cnt, hid):
    info = plsc.get_sparse_core_info()
    nc, ns = info.num_cores, info.num_subcores
    hh = hid // nc
    per_w = e_cnt // ns
    half_e = e_cnt // nc
    per_w_t = half_e // ns
    npad = zeros_h.shape[0]
    rows_per_tile = npad // ns
    mesh = plsc.VectorSubcoreMesh(core_axis_name="c", subcore_axis_name="s")

    @functools.partial(
        pl.kernel, mesh=mesh,
        compiler_params=pltpu.CompilerParams(use_tc_tiling_on_sc=False),
        out_type=(jax.ShapeDtypeStruct((nc, npad, hh), F32),
                  jax.ShapeDtypeStruct((nc, npad, XP), F32)),
        scratch_types=[pltpu.VMEM((2, CH), jnp.int32),
                       pltpu.VMEM((2, CH, hh), F32),
                       pltpu.VMEM((2, CH, XP), F32),
                       pltpu.VMEM_SHARED((npad, hh), F32),
                       pltpu.VMEM_SHARED((npad, XP), F32),
                       pltpu.SemaphoreType.DMA,
                       pltpu.SemaphoreType.DMA,
                       pltpu.SemaphoreType.DMA,
                       pltpu.SemaphoreType.DMA],
    )
    def k(m_h, t_h, row_h, z_h, zx_h, agg_h, dx_h,
          idx2, m2, t2, acc_s, accx_s, l0, l1, s0, s1):
        cid = lax.axis_index("c")
        sid = lax.axis_index("s")
        ls, ss = (l0, l1), (s0, s1)

        @pl.when(sid == 0)
        def _zero():
            pltpu.sync_copy(z_h, acc_s)
            pltpu.sync_copy(zx_h, accx_s)

        plsc.subcore_barrier()

        def add_loop(load_start, load_wait, buf2, acc, base, n_chunks):
            def add_wait(buf):
                pltpu.make_async_copy(buf2.at[buf], acc.at[idx2.at[buf]],
                                      ss[buf]).wait()

            def step(c, buf):
                oth = 1 - buf

                @pl.when(c >= 2)
                def _():
                    add_wait(buf)

                pltpu.sync_copy(row_h.at[pl.ds(base + c * CH, CH)],
                                idx2.at[buf])
                load_start(c, buf)

                @pl.when(c >= 1)
                def _():
                    load_wait(c - 1, oth)
                    pltpu.async_copy(buf2.at[oth], acc.at[idx2.at[oth]],
                                     ss[oth], add=True)

            def body(c, _):
                @pl.when(c % 2 == 0)
                def _():
                    step(c, 0)

                @pl.when(c % 2 == 1)
                def _():
                    step(c, 1)
                return 0

            lax.fori_loop(0, n_chunks, body, 0)
            lastb = (n_chunks - 1) % 2
            load_wait(n_chunks - 1, lastb)
            pltpu.async_copy(buf2.at[lastb], acc.at[idx2.at[lastb]],
                             ss[lastb], add=True)
            add_wait(1 - lastb)
            add_wait(lastb)

        m_base = sid * per_w

        def load_m(c, buf):
            pltpu.async_copy(
                m_h.at[pl.ds(m_base + c * CH, CH), pl.ds(cid * hh, hh)],
                m2.at[buf], ls[buf])

        def wait_m(c, buf):
            pltpu.make_async_copy(
                m_h.at[pl.ds(m_base + c * CH, CH), pl.ds(cid * hh, hh)],
                m2.at[buf], ls[buf]).wait()

        add_loop(load_m, wait_m, m2, acc_s, m_base, per_w // CH)

        t_base = cid * half_e + sid * per_w_t

        def load_t(c, buf):
            pltpu.async_copy(t_h.at[pl.ds(t_base + c * CH, CH)],
                             t2.at[buf], ls[buf])

        def wait_t(c, buf):
            pltpu.make_async_copy(t_h.at[pl.ds(t_base + c * CH, CH)],
                                  t2.at[buf], ls[buf]).wait()

        add_loop(load_t, wait_t, t2, accx_s, t_base, per_w_t // CH)

        plsc.subcore_barrier()

        r0 = sid * rows_per_tile
        pltpu.sync_copy(acc_s.at[pl.ds(r0, rows_per_tile)],
                        agg_h.at[cid, pl.ds(r0, rows_per_tile)])
        pltpu.sync_copy(accx_s.at[pl.ds(r0, rows_per_tile)],
                        dx_h.at[cid, pl.ds(r0, rows_per_tile)])

    return k(m, trans, row, zeros_h, zeros_x)


def kernel(h, x, edges, edge_attr, params):
    n, in_node = h.shape
    e_cnt = edges.shape[1]
    hid = params['emb_node'][0].shape[1]
    row = edges[0].astype(jnp.int32)
    col = edges[1].astype(jnp.int32)
    xp = jnp.zeros((n, XP), F32).at[:, :3].set(x)
    npad = ((n + 127) // 128) * 128
    zeros_h = jnp.zeros((npad, hid // 2), F32)
    zeros_x = jnp.zeros((npad, XP), F32)

    layers = params['layers']

    def split_w1(lp):
        w1, b1 = lp['edge_mlp1']
        return (w1[:hid], w1[hid:2 * hid], w1[2 * hid:2 * hid + 1],
                w1[2 * hid + 1:], b1.reshape(1, hid))

    w1a0, w1b0, wr0, w1e0, b10 = split_w1(layers[0])
    we, be = params['emb_node']
    wee, bee = params['emb_edge']

    h0, hw1, hw2, wfea, bfea = _tc_init(
        h, we, be.reshape(1, hid), w1a0, w1b0, b10,
        wee, w1e0, bee.reshape(1, hid), n, hid)

    hcur, xpcur = h0, xp
    e_feat = edge_attr
    for li, lp in enumerate(layers):
        first = li == 0
        last = li == len(layers) - 1
        w1a, w1b, wr, w1e, b1 = split_w1(lp)
        w2, b2 = lp['edge_mlp2']
        wc1, bc1 = lp['coord_mlp1']
        wc2, bc2 = lp['coord_mlp2']
        ghr, ghc, gxr, gxc = _sc_gather(hw1, hw2, xpcur, row, col, n, e_cnt, hid)
        bc2v = jnp.broadcast_to(bc2.reshape(1, 1), (1, hid))
        if first:
            wts = [wfea, bfea, wr, w2, b2.reshape(1, hid),
                   wc1, bc1.reshape(1, hid), wc2.reshape(1, hid), bc2v]
        else:
            wts = [w1e, wr, w2, b2.reshape(1, hid),
                   wc1, bc1.reshape(1, hid), wc2.reshape(1, hid), bc2v]
        if last:
            weo, beo = params['emb_edge_out']
            wts += [weo, beo.reshape(1, 16)]
        eouts = _tc_edge(first, last, ghr, ghc, e_feat, gxr, gxc, wts, e_cnt, hid)
        m = eouts[0]
        trans = eouts[1]
        agg, dx = _sc_scatter(m, trans, row, zeros_h, zeros_x, n, e_cnt, hid)
        wn1, bn1 = lp['node_mlp1']
        wn2, bn2 = lp['node_mlp2']
        hh = hid // 2
        if last:
            who, bho = params['emb_node_out']
            nwts = [wn1[:hid], wn1[hid:hid + hh], wn1[hid + hh:], bn1.reshape(1, hid),
                    wn2, bn2.reshape(1, hid), who, who, bho.reshape(1, in_node)]
            hout, xpcur = _tc_node(True, hcur, agg, dx, xpcur, nwts, n, hid)
        else:
            w1a_n, w1b_n, _, _, b1_n = split_w1(layers[li + 1])
            nwts = [wn1[:hid], wn1[hid:hid + hh], wn1[hid + hh:], bn1.reshape(1, hid),
                    wn2, bn2.reshape(1, hid), w1a_n, w1b_n, b1_n]
            hcur, xpcur, hw1, hw2 = _tc_node(False, hcur, agg, dx, xpcur, nwts, n, hid)
        e_feat = m

    e_out = eouts[2]
    return (hout, xpcur[:, :3], e_out)
